# Initial kernel scaffold; baseline (speedup 1.0000x reference)
import jax
import jax.numpy as jnp
from jax.experimental import pallas as pl


def kernel(words, classes, noun_table, class_table, special_emb):
    raise NotImplementedError("write your pallas kernel here")


# speedup vs baseline: 1.2153x; 1.2153x over previous
import functools
import math

import jax
import jax.numpy as jnp
import numpy as np
from jax import lax
from jax.experimental import pallas as pl
from jax.experimental.pallas import tpu as pltpu
from jax.experimental.pallas import tpu_sc as plsc

VOCAB = 100000
D = 128
L_TOK = 24
B = 4096
MAX_LEN = 25


def _pe_const(max_len, d_model):
    position = np.arange(0, max_len, dtype=np.float32)[:, None]
    div_term = np.exp(
        np.arange(0, d_model, 2).astype(np.float32) * (-math.log(10000.0) / d_model)
    )
    pe = np.zeros((max_len, d_model), dtype=np.float32)
    pe[:, 0::2] = np.sin(position * div_term)
    pe[:, 1::2] = np.cos(position * div_term)
    return pe


_PE = _pe_const(MAX_LEN, D)

_INFO = plsc.get_sparse_core_info()
_NC = _INFO.num_cores
_NS = _INFO.num_subcores
_NW = _NC * _NS

_B_PER_W = B // _NW
_NB = 8
_CHUNKS = _B_PER_W // _NB
_TOK = _NB * L_TOK
_HALF = _TOK // 2


def _sc_body(words_hbm, sidx_hbm, noun_hbm, addt_hbm, spec_hbm, out_hbm,
             idx_v, sidx_v, gath_v, add_v, spec_v, sem_g, sem_w):
    wid = lax.axis_index("s") * _NC + lax.axis_index("c")
    pltpu.sync_copy(spec_hbm, spec_v)

    def chunk_body(k, carry):
        tok0 = wid * (_B_PER_W * L_TOK) + k * _TOK
        pltpu.sync_copy(words_hbm.at[pl.ds(tok0, _HALF)], idx_v.at[0])
        pltpu.sync_copy(words_hbm.at[pl.ds(tok0 + _HALF, _HALF)], idx_v.at[1])
        pltpu.sync_copy(sidx_hbm.at[pl.ds(tok0, _HALF)], sidx_v.at[0])
        pltpu.sync_copy(sidx_hbm.at[pl.ds(tok0 + _HALF, _HALF)], sidx_v.at[1])
        cps = [
            pltpu.async_copy(noun_hbm.at[idx_v.at[0]],
                             gath_v.at[pl.ds(0, _HALF)], sem_g),
            pltpu.async_copy(noun_hbm.at[idx_v.at[1]],
                             gath_v.at[pl.ds(_HALF, _HALF)], sem_g),
            pltpu.async_copy(addt_hbm.at[sidx_v.at[0]],
                             add_v.at[pl.ds(0, _HALF)], sem_g),
            pltpu.async_copy(addt_hbm.at[sidx_v.at[1]],
                             add_v.at[pl.ds(_HALF, _HALF)], sem_g),
        ]
        for cp in cps:
            cp.wait()

        def add_row(r, c2):
            for q in range(D // 16):
                sl = pl.ds(q * 16, 16)
                gath_v[r, sl] = gath_v[r, sl] + add_v[r, sl]
            return c2

        lax.fori_loop(0, _TOK, add_row, 0, unroll=4)

        wcps = []
        for j in range(_NB):
            brow = (wid * _B_PER_W + k * _NB + j) * MAX_LEN
            wcps.append(pltpu.async_copy(
                spec_v, out_hbm.at[pl.ds(brow, 1)], sem_w))
            wcps.append(pltpu.async_copy(
                gath_v.at[pl.ds(j * L_TOK, L_TOK)],
                out_hbm.at[pl.ds(brow + 1, L_TOK)], sem_w))
        for cp in wcps:
            cp.wait()
        return carry

    lax.fori_loop(0, _CHUNKS, chunk_body, 0)


def kernel(words, classes, noun_table, class_table, special_emb):
    words_flat = words.astype(jnp.int32).reshape(-1)
    sidx = (2 * jnp.arange(L_TOK, dtype=jnp.int32)[None, :]
            + classes.astype(jnp.int32)).reshape(-1)
    pe = jnp.asarray(_PE[:L_TOK])
    addt = (pe[:, None, :] + class_table[None, :, :]).reshape(2 * L_TOK, D)

    mesh = plsc.VectorSubcoreMesh(core_axis_name="c", subcore_axis_name="s")
    run = functools.partial(
        pl.kernel,
        mesh=mesh,
        compiler_params=pltpu.CompilerParams(use_tc_tiling_on_sc=False),
        out_type=jax.ShapeDtypeStruct((B * MAX_LEN, D), jnp.float32),
        scratch_types=[
            pltpu.VMEM((2, _HALF), jnp.int32),
            pltpu.VMEM((2, _HALF), jnp.int32),
            pltpu.VMEM((_TOK, D), jnp.float32),
            pltpu.VMEM((_TOK, D), jnp.float32),
            pltpu.VMEM((1, D), jnp.float32),
            pltpu.SemaphoreType.DMA,
            pltpu.SemaphoreType.DMA,
        ],
    )(_sc_body)
    out = run(words_flat, sidx, noun_table, addt, special_emb)
    return out.reshape(B, MAX_LEN, D)
